# baseline (speedup 1.0000x reference)
import functools

import jax
import jax.numpy as jnp
from jax import lax
from jax.experimental import pallas as pl
from jax.experimental.pallas import tpu as pltpu
from jax.experimental.pallas import tpu_sc as plsc

_N = 8192
_A = 1.1201
_B = 0.7990

_NC = 2
_NS = 16
_NW = _NC * _NS
_TOTAL = 16384
_BPW = _TOTAL // _NW
_CHUNK = 128
_NCHUNK = _BPW // _CHUNK
_NLINES = _N * _N // 16


@functools.lru_cache(maxsize=None)
def _make_sc_gather():
    mesh = plsc.VectorSubcoreMesh(core_axis_name="c", subcore_axis_name="s")

    @functools.partial(
        pl.kernel,
        mesh=mesh,
        out_type=jax.ShapeDtypeStruct((_TOTAL,), jnp.float32),
        compiler_params=pltpu.CompilerParams(
            needs_layout_passes=False, use_tc_tiling_on_sc=False),
        scratch_types=[
            pltpu.VMEM((_BPW,), jnp.int32),
            pltpu.VMEM((_BPW,), jnp.int32),
            pltpu.VMEM((_NCHUNK, _CHUNK), jnp.int32),
            pltpu.VMEM((_BPW,), jnp.int32),
            pltpu.VMEM((_BPW, 16), jnp.float32),
            pltpu.VMEM((_BPW,), jnp.float32),
            pltpu.SemaphoreType.DMA,
            pltpu.SemaphoreType.DMA,
            pltpu.SemaphoreType.DMA,
        ],
    )
    def _sc_gather(lines_hbm, ind1_hbm, ind2_hbm, high_hbm, i1_v, i2_v, q_v,
                   lane_v, vals_v, out_v, sem, sem_idx, sem_out):
        wid = lax.axis_index("s") * _NC + lax.axis_index("c")
        base = wid * _BPW
        idx1 = pltpu.make_async_copy(ind1_hbm.at[pl.ds(base, _BPW)], i1_v,
                                     sem_idx)
        idx2 = pltpu.make_async_copy(ind2_hbm.at[pl.ds(base, _BPW)], i2_v,
                                     sem_idx)
        idx1.start()
        idx2.start()
        idx1.wait()
        idx2.wait()

        lane16 = lax.iota(jnp.int32, 16)

        def math_chunk(j, _):
            def math_vec(k, _):
                off = pl.multiple_of(j * _CHUNK + k * 16, 16)
                sl = pl.ds(off, 16)
                row = i1_v[sl]
                col = i2_v[sl]
                q = (((row >> 3) * (_N // 128) + (col >> 7)) * 64
                     + ((row & 7) << 3) + ((col >> 4) & 7))
                q_v[j, pl.ds(pl.multiple_of(k * 16, 16), 16)] = q
                lane_v[sl] = col & 15
                return _

            lax.fori_loop(0, _CHUNK // 16, math_vec, 0)
            pltpu.async_copy(
                lines_hbm.at[q_v.at[j]],
                vals_v.at[pl.ds(pl.multiple_of(j * _CHUNK, _CHUNK), _CHUNK)],
                sem)
            return _

        lax.fori_loop(0, _NCHUNK, math_chunk, 0)

        def wait_chunk(j, _):
            coff = pl.multiple_of(j * _CHUNK, _CHUNK)
            pltpu.make_async_copy(
                lines_hbm.at[q_v.at[j]],
                vals_v.at[pl.ds(coff, _CHUNK)], sem).wait()
            return _

        lax.fori_loop(0, _NCHUNK, wait_chunk, 0)

        def extract_vec(k, _):
            off = pl.multiple_of(k * 16, 16)
            sl = pl.ds(off, 16)
            rows16 = lane16 + off
            out_v[sl] = plsc.load_gather(vals_v, [rows16, lane_v[sl]])
            return _

        lax.fori_loop(0, _BPW // 16, extract_vec, 0)
        pltpu.sync_copy(out_v, high_hbm.at[pl.ds(base, _BPW)])

    return _sc_gather


def _lowdim_body(x1_ref, y1_ref, x2_ref, y2_ref, out_ref):
    dx = x1_ref[...] - x2_ref[...]
    dy = y1_ref[...] - y2_ref[...]
    s = dx * dx + dy * dy
    powed = jnp.exp(_B * jnp.log(s))
    out_ref[...] = 1.0 / (1.0 + _A * powed)


def kernel(p1, p2, ind1, ind2, sym_prob):
    lines = sym_prob.reshape(1024, 8, 64, 128).transpose(0, 2, 1, 3) \
        .reshape(_NLINES, 16)
    high = _make_sc_gather()(lines, ind1.astype(jnp.int32),
                             ind2.astype(jnp.int32))

    x1 = p1[:, 0].reshape(128, 128)
    y1 = p1[:, 1].reshape(128, 128)
    x2 = p2[:, 0].reshape(128, 128)
    y2 = p2[:, 1].reshape(128, 128)
    low = pl.pallas_call(
        _lowdim_body,
        out_shape=jax.ShapeDtypeStruct((128, 128), jnp.float32),
    )(x1, y1, x2, y2).reshape(_TOTAL)
    return (low, high)


# speedup vs baseline: 1.0088x; 1.0088x over previous
import functools

import jax
import jax.numpy as jnp
from jax import lax
from jax.experimental import pallas as pl
from jax.experimental.pallas import tpu as pltpu
from jax.experimental.pallas import tpu_sc as plsc

_N = 8192
_A = 1.1201
_B = 0.7990

_NC = 2
_NS = 16
_NW = _NC * _NS
_TOTAL = 16384
_BPW = _TOTAL // _NW
_CHUNK = 128
_NCHUNK = _BPW // _CHUNK
_NLINES = _N * _N // 16


@functools.lru_cache(maxsize=None)
def _make_sc_gather():
    mesh = plsc.VectorSubcoreMesh(core_axis_name="c", subcore_axis_name="s")

    @functools.partial(
        pl.kernel,
        mesh=mesh,
        out_type=jax.ShapeDtypeStruct((_TOTAL,), jnp.float32),
        compiler_params=pltpu.CompilerParams(
            needs_layout_passes=False, use_tc_tiling_on_sc=False),
        scratch_types=[
            pltpu.VMEM((_BPW,), jnp.int32),
            pltpu.VMEM((_BPW,), jnp.int32),
            pltpu.VMEM((_NCHUNK, _CHUNK), jnp.int32),
            pltpu.VMEM((_BPW,), jnp.int32),
            pltpu.VMEM((_BPW, 16), jnp.float32),
            pltpu.VMEM((_BPW,), jnp.float32),
            pltpu.SemaphoreType.DMA,
            pltpu.SemaphoreType.DMA,
            pltpu.SemaphoreType.DMA,
        ],
    )
    def _sc_gather(lines_hbm, ind1_hbm, ind2_hbm, high_hbm, i1_v, i2_v, q_v,
                   lane_v, vals_v, out_v, sem, sem_idx, sem_out):
        wid = lax.axis_index("s") * _NC + lax.axis_index("c")
        base = wid * _BPW
        idx1 = pltpu.make_async_copy(ind1_hbm.at[pl.ds(base, _BPW)], i1_v,
                                     sem_idx)
        idx2 = pltpu.make_async_copy(ind2_hbm.at[pl.ds(base, _BPW)], i2_v,
                                     sem_idx)
        idx1.start()
        idx2.start()
        idx1.wait()
        idx2.wait()

        lane16 = lax.iota(jnp.int32, 16)

        def math_chunk(j, _):
            def math_vec(k, _):
                off = pl.multiple_of(j * _CHUNK + k * 16, 16)
                sl = pl.ds(off, 16)
                row = i1_v[sl]
                col = i2_v[sl]
                q = (((row >> 3) * (_N // 128) + (col >> 7)) * 64
                     + ((row & 7) << 3) + ((col >> 4) & 7))
                q_v[j, pl.ds(pl.multiple_of(k * 16, 16), 16)] = q
                lane_v[sl] = col & 15
                return _

            lax.fori_loop(0, _CHUNK // 16, math_vec, 0)
            pltpu.async_copy(
                lines_hbm.at[q_v.at[j]],
                vals_v.at[pl.ds(pl.multiple_of(j * _CHUNK, _CHUNK), _CHUNK)],
                sem)
            return _

        lax.fori_loop(0, _NCHUNK, math_chunk, 0)

        def extract_chunk(j, _):
            coff = pl.multiple_of(j * _CHUNK, _CHUNK)
            pltpu.make_async_copy(
                lines_hbm.at[q_v.at[j]],
                vals_v.at[pl.ds(coff, _CHUNK)], sem).wait()

            def extract_vec(k, _):
                off = pl.multiple_of(j * _CHUNK + k * 16, 16)
                sl = pl.ds(off, 16)
                rows16 = lane16 + off
                out_v[sl] = plsc.load_gather(vals_v, [rows16, lane_v[sl]])
                return _

            lax.fori_loop(0, _CHUNK // 16, extract_vec, 0)
            pltpu.async_copy(
                out_v.at[pl.ds(coff, _CHUNK)],
                high_hbm.at[pl.ds(base + coff, _CHUNK)], sem_out)
            return _

        lax.fori_loop(0, _NCHUNK, extract_chunk, 0)

        def drain_chunk(j, _):
            coff = pl.multiple_of(j * _CHUNK, _CHUNK)
            pltpu.make_async_copy(
                out_v.at[pl.ds(coff, _CHUNK)],
                high_hbm.at[pl.ds(base + coff, _CHUNK)], sem_out).wait()
            return _

        lax.fori_loop(0, _NCHUNK, drain_chunk, 0)

    return _sc_gather


def _lowdim_body(x1_ref, y1_ref, x2_ref, y2_ref, out_ref):
    dx = x1_ref[...] - x2_ref[...]
    dy = y1_ref[...] - y2_ref[...]
    s = dx * dx + dy * dy
    powed = jnp.exp(_B * jnp.log(s))
    out_ref[...] = 1.0 / (1.0 + _A * powed)


def kernel(p1, p2, ind1, ind2, sym_prob):
    lines = sym_prob.reshape(1024, 8, 64, 128).transpose(0, 2, 1, 3) \
        .reshape(_NLINES, 16)
    high = _make_sc_gather()(lines, ind1.astype(jnp.int32),
                             ind2.astype(jnp.int32))

    x1 = p1[:, 0].reshape(128, 128)
    y1 = p1[:, 1].reshape(128, 128)
    x2 = p2[:, 0].reshape(128, 128)
    y2 = p2[:, 1].reshape(128, 128)
    low = pl.pallas_call(
        _lowdim_body,
        out_shape=jax.ShapeDtypeStruct((128, 128), jnp.float32),
    )(x1, y1, x2, y2).reshape(_TOTAL)
    return (low, high)


# speedup vs baseline: 1.0207x; 1.0118x over previous
import functools

import jax
import jax.numpy as jnp
from jax import lax
from jax.experimental import pallas as pl
from jax.experimental.pallas import tpu as pltpu
from jax.experimental.pallas import tpu_sc as plsc

_N = 8192
_A = 1.1201
_B = 0.7990

_NC = 2
_NS = 16
_NW = _NC * _NS
_TOTAL = 16384
_BPW = _TOTAL // _NW
_CHUNK = 128
_NCHUNK = _BPW // _CHUNK
_NLINES = _N * _N // 16


@functools.lru_cache(maxsize=None)
def _make_sc_gather():
    mesh = plsc.VectorSubcoreMesh(core_axis_name="c", subcore_axis_name="s")

    @functools.partial(
        pl.kernel,
        mesh=mesh,
        out_type=jax.ShapeDtypeStruct((_TOTAL,), jnp.float32),
        compiler_params=pltpu.CompilerParams(
            needs_layout_passes=False, use_tc_tiling_on_sc=False),
        scratch_types=[
            pltpu.VMEM((_BPW,), jnp.int32),
            pltpu.VMEM((_BPW,), jnp.int32),
            pltpu.VMEM((_NCHUNK, _CHUNK), jnp.int32),
            pltpu.VMEM((_BPW, 16), jnp.float32),
            pltpu.VMEM((_BPW,), jnp.float32),
            pltpu.SemaphoreType.DMA,
            pltpu.SemaphoreType.DMA,
            pltpu.SemaphoreType.DMA,
        ],
    )
    def _sc_gather(lines_hbm, ind1_hbm, ind2_hbm, high_hbm, i1_v, i2_v, q_v,
                   vals_v, out_v, sem, sem_idx, sem_out):
        wid = lax.axis_index("s") * _NC + lax.axis_index("c")
        base = wid * _BPW
        idx1 = pltpu.make_async_copy(ind1_hbm.at[pl.ds(base, _BPW)], i1_v,
                                     sem_idx)
        idx2 = pltpu.make_async_copy(ind2_hbm.at[pl.ds(base, _BPW)], i2_v,
                                     sem_idx)
        idx1.start()
        idx2.start()
        idx1.wait()
        idx2.wait()

        lane16 = lax.iota(jnp.int32, 16)

        def math_chunk(j, _):
            def math_vec(k, _):
                off = pl.multiple_of(j * _CHUNK + k * 16, 16)
                sl = pl.ds(off, 16)
                row = i1_v[sl]
                col = i2_v[sl]
                q = (((row >> 3) * (_N // 128) + (col >> 7)) * 64
                     + ((row & 7) << 3) + ((col >> 4) & 7))
                q_v[j, pl.ds(pl.multiple_of(k * 16, 16), 16)] = q
                return _

            lax.fori_loop(0, _CHUNK // 16, math_vec, 0)
            pltpu.async_copy(
                lines_hbm.at[q_v.at[j]],
                vals_v.at[pl.ds(pl.multiple_of(j * _CHUNK, _CHUNK), _CHUNK)],
                sem)
            return _

        lax.fori_loop(0, _NCHUNK, math_chunk, 0)

        def extract_chunk(j, _):
            coff = pl.multiple_of(j * _CHUNK, _CHUNK)
            pltpu.make_async_copy(
                lines_hbm.at[q_v.at[j]],
                vals_v.at[pl.ds(coff, _CHUNK)], sem).wait()

            def extract_vec(k, _):
                off = pl.multiple_of(j * _CHUNK + k * 16, 16)
                sl = pl.ds(off, 16)
                rows16 = lane16 + off
                out_v[sl] = plsc.load_gather(vals_v,
                                             [rows16, i2_v[sl] & 15])
                return _

            lax.fori_loop(0, _CHUNK // 16, extract_vec, 0)
            pltpu.async_copy(
                out_v.at[pl.ds(coff, _CHUNK)],
                high_hbm.at[pl.ds(base + coff, _CHUNK)], sem_out)
            return _

        lax.fori_loop(0, _NCHUNK, extract_chunk, 0)

        pltpu.make_async_copy(out_v, high_hbm.at[pl.ds(base, _BPW)],
                              sem_out).wait()

    return _sc_gather


def _lowdim_body(x1_ref, y1_ref, x2_ref, y2_ref, out_ref):
    dx = x1_ref[...] - x2_ref[...]
    dy = y1_ref[...] - y2_ref[...]
    s = dx * dx + dy * dy
    powed = jnp.exp(_B * jnp.log(s))
    out_ref[...] = 1.0 / (1.0 + _A * powed)


def kernel(p1, p2, ind1, ind2, sym_prob):
    lines = sym_prob.reshape(1024, 8, 64, 128).transpose(0, 2, 1, 3) \
        .reshape(_NLINES, 16)
    high = _make_sc_gather()(lines, ind1.astype(jnp.int32),
                             ind2.astype(jnp.int32))

    x1 = p1[:, 0].reshape(128, 128)
    y1 = p1[:, 1].reshape(128, 128)
    x2 = p2[:, 0].reshape(128, 128)
    y2 = p2[:, 1].reshape(128, 128)
    low = pl.pallas_call(
        _lowdim_body,
        out_shape=jax.ShapeDtypeStruct((128, 128), jnp.float32),
    )(x1, y1, x2, y2).reshape(_TOTAL)
    return (low, high)
